# baseline (speedup 1.0000x reference)
import jax
import jax.numpy as jnp
from jax.experimental import pallas as pl

_B, _C = 16384, 1000
_K = 12288
_CHI, _IHI, _CLO, _ILO = 125, 128, 8, 128
_BI = 16
_NBLK = _IHI // _BI

_INT_MIN = -2147483648


def _loss_body(x_ref, t_ref, loss_ref):
    x = x_ref[...]
    t = t_ref[...]
    m1 = jnp.max(x, axis=0)
    m = jnp.max(m1, axis=1)
    e = jnp.exp(x - m[None, :, None, :])
    s = jnp.sum(jnp.sum(e, axis=0), axis=1)
    cls = jax.lax.broadcasted_iota(jnp.int32, (_CHI, _BI, _CLO, _ILO), 0) * _CLO \
        + jax.lax.broadcasted_iota(jnp.int32, (_CHI, _BI, _CLO, _ILO), 2)
    mask = cls == t[None, :, None, :]
    picked = jnp.sum(jnp.sum(jnp.where(mask, x, 0.0), axis=0), axis=1)
    loss_ref[...] = jnp.log(s) + m - picked


def _topk_body(loss_ref, out_ref):
    x = loss_ref[...]
    bits = jax.lax.bitcast_convert_type(x, jnp.int32)
    key = jnp.where(bits >= 0, bits, bits ^ jnp.int32(0x7FFFFFFF))

    def body(r, T):
        sh = jnp.int32(28) - 4 * r
        digit = jnp.int32(0)
        for j in range(1, 16):
            cand = T | jax.lax.shift_left(jnp.int32(j), sh)
            scand = cand ^ jnp.int32(_INT_MIN)
            cnt = jnp.sum((key >= scand).astype(jnp.int32))
            digit += (cnt >= _K).astype(jnp.int32)
        return T | jax.lax.shift_left(digit, sh)

    T = jax.lax.fori_loop(0, 8, body, jnp.int32(0))
    kth = T ^ jnp.int32(_INT_MIN)

    gt = key > kth
    cnt_gt = jnp.sum(gt.astype(jnp.int32))
    sum_gt = jnp.sum(jnp.where(gt, x, 0.0))
    tval = jnp.max(jnp.where(key == kth, x, -jnp.inf))
    res = (sum_gt + (_K - cnt_gt).astype(jnp.float32) * tval) / _K
    out_ref[...] = jnp.full((1, 1), res, jnp.float32)


def kernel(input, target):
    x4 = input.T.reshape(_CHI, _CLO, _IHI, _ILO).transpose(0, 2, 1, 3)
    t2 = target.astype(jnp.int32).reshape(_IHI, _ILO)

    loss = pl.pallas_call(
        _loss_body,
        grid=(_NBLK,),
        in_specs=[
            pl.BlockSpec((_CHI, _BI, _CLO, _ILO), lambda b: (0, b, 0, 0)),
            pl.BlockSpec((_BI, _ILO), lambda b: (b, 0)),
        ],
        out_specs=pl.BlockSpec((_BI, _ILO), lambda b: (b, 0)),
        out_shape=jax.ShapeDtypeStruct((_IHI, _ILO), jnp.float32),
    )(x4, t2)

    out = pl.pallas_call(
        _topk_body,
        out_shape=jax.ShapeDtypeStruct((1, 1), jnp.float32),
    )(loss)
    return out[0, 0]
